# baseline (speedup 1.0000x reference)
import functools
import jax
import jax.numpy as jnp
import numpy as np
from jax import lax
from jax.experimental import pallas as pl
from jax.experimental.pallas import tpu as pltpu
from jax.experimental.pallas import tpu_sc as plsc

_Y_LOC = 100
_X_LOC = 100
_T_H = 96
_T_W = 96

_BR0 = 96
_BRN = 104
_BCN = 256

_P = 12
_RB = 8


def _band_cast_body(x_ref, o_ref):
    o_ref[...] = x_ref[...].astype(jnp.uint8)


def _tc_band_cast(x):
    n, h, w = x.shape
    return pl.pallas_call(
        _band_cast_body,
        grid=(n // _P, _BRN // _RB),
        in_specs=[pl.BlockSpec(
            (_P, _RB, _BCN), lambda i, j: (i, j + _BR0 // _RB, 0))],
        out_specs=pl.BlockSpec((_P, _RB, _BCN), lambda i, j: (i, j, 0)),
        out_shape=jax.ShapeDtypeStruct((n, _BRN, _BCN), jnp.uint8),
    )(x)


def _sc_erase_band(yband):
    n = yband.shape[0]
    planes_per_worker = n // 32
    mesh = plsc.VectorSubcoreMesh(
        core_axis_name="c", subcore_axis_name="s", num_cores=2,
        num_subcores=16)

    m_lo_arr = jnp.asarray(
        np.repeat((np.arange(16) < _X_LOC - 96)[None, :], 4, 0)
        .astype(np.uint8))
    m_hi_arr = jnp.asarray(
        np.repeat((np.arange(16) + 192 >= _X_LOC + _T_W)[None, :], 4, 0)
        .astype(np.uint8))

    @functools.partial(
        pl.kernel,
        out_type=jax.ShapeDtypeStruct((n, _BRN, _BCN), jnp.uint8),
        mesh=mesh,
        scratch_types=[
            pltpu.VMEM((_BRN, _BCN), jnp.uint8),
            pltpu.VMEM((4, 16), jnp.uint8),
            pltpu.VMEM((4, 16), jnp.uint8),
        ],
    )
    def k(y_hbm, mlo_hbm, mhi_hbm, band_hbm, bv, mlo_v, mhi_v):
        wid = lax.axis_index("s") * 2 + lax.axis_index("c")
        pltpu.sync_copy(mlo_hbm, mlo_v)
        pltpu.sync_copy(mhi_hbm, mhi_v)

        for i in range(planes_per_worker):
            p = wid * planes_per_worker + i
            pltpu.sync_copy(y_hbm.at[p], bv)

            def zgrp(g, carry):
                r = pl.multiple_of(_Y_LOC - _BR0 + 4 * g, 4)
                m_lo = mlo_v[pl.ds(0, 4), pl.ds(0, 16)]
                m_hi = mhi_v[pl.ds(0, 4), pl.ds(0, 16)]
                v0 = bv[pl.ds(r, 4), pl.ds(96, 16)]
                bv[pl.ds(r, 4), pl.ds(96, 16)] = v0 * m_lo
                for kk in range(1, 6):
                    v = bv[pl.ds(r, 4), pl.ds(96 + 16 * kk, 16)]
                    bv[pl.ds(r, 4), pl.ds(96 + 16 * kk, 16)] = (
                        v * jnp.uint8(0))
                v6 = bv[pl.ds(r, 4), pl.ds(192, 16)]
                bv[pl.ds(r, 4), pl.ds(192, 16)] = v6 * m_hi
                return carry

            lax.fori_loop(0, _T_H // 4, zgrp, 0)
            pltpu.sync_copy(bv, band_hbm.at[p])

    return k(yband, m_lo_arr, m_hi_arr)


def _merge_body(x_ref, b_ref, o_ref):
    o_ref[...] = x_ref[...].astype(jnp.uint8)
    o_ref[:, _BR0:_BR0 + _BRN, 0:_BCN] = b_ref[...]


def _tc_merge(x, band):
    n, h, w = x.shape
    return pl.pallas_call(
        _merge_body,
        grid=(n // _P,),
        in_specs=[
            pl.BlockSpec((_P, h, w), lambda i: (i, 0, 0)),
            pl.BlockSpec((_P, _BRN, _BCN), lambda i: (i, 0, 0)),
        ],
        out_specs=pl.BlockSpec((_P, h, w), lambda i: (i, 0, 0)),
        out_shape=jax.ShapeDtypeStruct((n, h, w), jnp.uint8),
    )(x, band)


def kernel(inputs):
    b, h, w, c = inputs.shape
    x = jnp.transpose(inputs, (0, 3, 1, 2)).reshape(b * c, h, w)
    yband = _tc_band_cast(x)
    band = _sc_erase_band(yband)
    out = _tc_merge(x, band)
    return jnp.transpose(out.reshape(b, c, h, w), (0, 2, 3, 1))


# speedup vs baseline: 4.4942x; 4.4942x over previous
import jax
import jax.numpy as jnp
from jax.experimental import pallas as pl

_Y_LOC = 100
_X_LOC = 100
_T_H = 96
_T_W = 96


_P = 12


def _erase_body(x_ref, o_ref):
    o_ref[...] = x_ref[...].astype(jnp.uint8)
    o_ref[:, _Y_LOC:_Y_LOC + _T_H, _X_LOC:_X_LOC + _T_W] = (
        jnp.zeros((_P, _T_H, _T_W), jnp.uint8))


def kernel(inputs):
    b, h, w, c = inputs.shape
    x = jnp.transpose(inputs, (0, 3, 1, 2)).reshape(b * c, h, w)
    out = pl.pallas_call(
        _erase_body,
        grid=(b * c // _P,),
        in_specs=[pl.BlockSpec((_P, h, w), lambda i: (i, 0, 0))],
        out_specs=pl.BlockSpec((_P, h, w), lambda i: (i, 0, 0)),
        out_shape=jax.ShapeDtypeStruct((b * c, h, w), jnp.uint8),
    )(x)
    return jnp.transpose(out.reshape(b, c, h, w), (0, 2, 3, 1))
